# Initial kernel scaffold; baseline (speedup 1.0000x reference)
import jax
import jax.numpy as jnp
from jax.experimental import pallas as pl


def kernel(loc_preds, conf_preds, gt_boxes, gt_labels, default_boxes):
    raise NotImplementedError("write your pallas kernel here")


# speedup vs baseline: 19.8029x; 19.8029x over previous
import functools

import jax
import jax.numpy as jnp
from jax import lax
from jax.experimental import pallas as pl
from jax.experimental.pallas import tpu as pltpu

NUM_CLASSES = 21
ALPHA = 0.25
GAMMA = 1.5
LAMBDA_LOC = 1.5
LAMBDA_CONF = 1.0
VAR = (0.1, 0.1)
THRESHOLD = 0.4
NEG_POS_RATIO = 2

_CW2 = (3, 4, 5, 9, 10, 11, 16, 17, 18)
_CW075 = (15, 7, 12, 8)


def _ssd_kernel(gtb_ref, gtl_ref, def_ref, loc_ref, conf_ref, out_ref, *, P, G):
    b = pl.program_id(0)
    R = def_ref.shape[1]
    L = def_ref.shape[2]

    shape = (R, L)
    lin = (lax.broadcasted_iota(jnp.int32, shape, 0) * L
           + lax.broadcasted_iota(jnp.int32, shape, 1))
    valid = lin < P

    dx1 = def_ref[0]
    dy1 = def_ref[1]
    dx2 = def_ref[2]
    dy2 = def_ref[3]
    dw = dx2 - dx1
    dh = dy2 - dy1
    dcx = dx1 + dw * 0.5
    dcy = dy1 + dh * 0.5
    darea = dw * dh

    tx1 = [gtb_ref[0, g, 0] for g in range(G)]
    ty1 = [gtb_ref[0, g, 1] for g in range(G)]
    tx2 = [gtb_ref[0, g, 2] for g in range(G)]
    ty2 = [gtb_ref[0, g, 3] for g in range(G)]
    tlab = [gtl_ref[0, 0, g] for g in range(G)]

    bto = None
    bti = None
    bp = []
    big = jnp.int32(0x7FFFFFFF)
    for g in range(G):
        tarea = (tx2[g] - tx1[g]) * (ty2[g] - ty1[g])
        iw = jnp.minimum(dx2, tx2[g]) - jnp.maximum(dx1, tx1[g])
        ih = jnp.minimum(dy2, ty2[g]) - jnp.maximum(dy1, ty1[g])
        inter = jnp.maximum(iw, 0.0) * jnp.maximum(ih, 0.0)
        ov = inter / (darea + tarea - inter)
        m = jnp.max(ov)
        bp.append(jnp.min(jnp.where(ov == m, lin, big)))
        if g == 0:
            bto, bti = ov, jnp.zeros(shape, jnp.int32)
        else:
            take = ov > bto
            bti = jnp.where(take, g, bti)
            bto = jnp.where(take, ov, bto)

    for g in range(G):
        hit = lin == bp[g]
        bti = jnp.where(hit, g, bti)
        bto = jnp.where(hit, 2.0, bto)

    gx1 = jnp.zeros(shape, jnp.float32)
    gy1 = jnp.zeros(shape, jnp.float32)
    gx2 = jnp.zeros(shape, jnp.float32)
    gy2 = jnp.zeros(shape, jnp.float32)
    glab = jnp.zeros(shape, jnp.int32)
    for g in range(G):
        hit = bti == g
        gx1 = jnp.where(hit, tx1[g], gx1)
        gy1 = jnp.where(hit, ty1[g], gy1)
        gx2 = jnp.where(hit, tx2[g], gx2)
        gy2 = jnp.where(hit, ty2[g], gy2)
        glab = jnp.where(hit, tlab[g], glab)

    conf_t = jnp.where(bto < THRESHOLD, 0, glab)
    pos = jnp.logical_and(conf_t > 0, valid)
    posf = pos.astype(jnp.float32)
    npos_f = jnp.sum(posf)
    npos_i = jnp.sum(pos.astype(jnp.int32))

    gw = gx2 - gx1
    gh = gy2 - gy1
    gcx = gx1 + gw * 0.5
    gcy = gy1 + gh * 0.5
    e0 = (gcx - dcx) / (dw * VAR[0] + 1e-8)
    e1 = (gcy - dcy) / (dh * VAR[0] + 1e-8)
    e2 = jnp.log(gw / (dw + 1e-8) + 1e-8) / VAR[1]
    e3 = jnp.log(gh / (dh + 1e-8) + 1e-8) / VAR[1]

    def sl1(x, t):
        d = jnp.abs(x - t)
        return jnp.where(d < 1.0, 0.5 * d * d, d - 0.5)

    sl = (sl1(loc_ref[0, 0], e0) + sl1(loc_ref[0, 1], e1)
          + sl1(loc_ref[0, 2], e2) + sl1(loc_ref[0, 3], e3))
    sl_sum = jnp.sum(sl * posf)

    m = conf_ref[0, 0]
    for c in range(1, NUM_CLASSES):
        m = jnp.maximum(m, conf_ref[0, c])
    s = jnp.exp(conf_ref[0, 0] - m)
    for c in range(1, NUM_CLASSES):
        s = s + jnp.exp(conf_ref[0, c] - m)
    lse = m + jnp.log(s)

    logit_t = conf_ref[0, 0]
    for c in range(1, NUM_CLASSES):
        logit_t = jnp.where(conf_t == c, conf_ref[0, c], logit_t)
    ce = lse - logit_t
    pt = jnp.exp(-ce)
    omp = jnp.maximum(1.0 - pt, 0.0)
    cw = jnp.full(shape, 1.0, jnp.float32)
    for c in _CW2:
        cw = jnp.where(conf_t == c, 2.0, cw)
    for c in _CW075:
        cw = jnp.where(conf_t == c, 0.75, cw)
    cw = jnp.where(conf_t == 0, 0.5, cw)
    focal = ALPHA * omp * jnp.sqrt(omp) * cw * ce

    focal_pos_sum = jnp.sum(focal * posf)
    loss_c = jnp.where(jnp.logical_and(valid, jnp.logical_not(pos)), focal, 0.0)

    k = jnp.minimum(NEG_POS_RATIO * npos_i, P - 1)
    vi = lax.bitcast_convert_type(loss_c, jnp.int32)

    def body(_, lohi):
        lo, hi = lohi
        mid = lo + lax.div(hi - lo, 2)
        c = jnp.sum((vi >= mid).astype(jnp.int32))
        good = c >= k
        return jnp.where(good, mid, lo), jnp.where(good, hi, mid)

    lo, _ = lax.fori_loop(0, 31, body, (jnp.int32(0), jnp.int32(0x7F800000)))
    vk = lax.bitcast_convert_type(lo, jnp.float32)
    gt_mask = vi > lo
    cnt_gt = jnp.sum(gt_mask.astype(jnp.int32))
    topk_sum = (jnp.sum(jnp.where(gt_mask, loss_c, 0.0))
                + (k - cnt_gt).astype(jnp.float32) * vk)

    kf = k.astype(jnp.float32)
    sel_cnt = npos_f + jnp.minimum(kf, P - npos_f)

    @pl.when(b == 0)
    def _():
        out_ref[0] = 0.0
        out_ref[1] = 0.0
        out_ref[2] = 0.0
        out_ref[3] = 0.0

    out_ref[0] += sl_sum
    out_ref[1] += npos_f
    out_ref[2] += focal_pos_sum + topk_sum
    out_ref[3] += sel_cnt


def _ssd_loss_pallas(loc_preds, conf_preds, gt_boxes, gt_labels, default_boxes,
                     interpret=False):
    B, P, C = conf_preds.shape
    G = gt_boxes.shape[1]
    L = 128
    PP = ((P + L - 1) // L) * L
    R = PP // L

    locT = jnp.moveaxis(loc_preds, 2, 1)
    locT = jnp.pad(locT, ((0, 0), (0, 0), (0, PP - P))).reshape(B, 4, R, L)
    confT = jnp.moveaxis(conf_preds, 2, 1)
    confT = jnp.pad(confT, ((0, 0), (0, 0), (0, PP - P))).reshape(B, C, R, L)
    defT = jnp.pad(default_boxes.T, ((0, 0), (0, PP - P))).reshape(4, R, L)
    gtb = gt_boxes
    gtl = gt_labels.astype(jnp.int32).reshape(B, 1, G)

    acc = pl.pallas_call(
        functools.partial(_ssd_kernel, P=P, G=G),
        grid=(B,),
        in_specs=[
            pl.BlockSpec((1, G, 4), lambda b: (b, 0, 0),
                         memory_space=pltpu.SMEM),
            pl.BlockSpec((1, 1, G), lambda b: (b, 0, 0),
                         memory_space=pltpu.SMEM),
            pl.BlockSpec((4, R, L), lambda b: (0, 0, 0)),
            pl.BlockSpec((1, 4, R, L), lambda b: (b, 0, 0, 0)),
            pl.BlockSpec((1, C, R, L), lambda b: (b, 0, 0, 0)),
        ],
        out_specs=pl.BlockSpec((4,), lambda b: (0,), memory_space=pltpu.SMEM),
        out_shape=jax.ShapeDtypeStruct((4,), jnp.float32),
        interpret=interpret,
    )(gtb, gtl, defT, locT, confT)

    return LAMBDA_LOC * acc[0] / acc[1] + LAMBDA_CONF * acc[2] / acc[3]


def kernel(loc_preds, conf_preds, gt_boxes, gt_labels, default_boxes):
    return _ssd_loss_pallas(loc_preds, conf_preds, gt_boxes, gt_labels,
                            default_boxes)


# speedup vs baseline: 50.1150x; 2.5307x over previous
import functools

import jax
import jax.numpy as jnp
from jax import lax
from jax.experimental import pallas as pl
from jax.experimental.pallas import tpu as pltpu
from jax.experimental.pallas import tpu_sc as plsc

NUM_CLASSES = 21
ALPHA = 0.25
GAMMA = 1.5
LAMBDA_LOC = 1.5
LAMBDA_CONF = 1.0
VAR = (0.1, 0.1)
THRESHOLD = 0.4
NEG_POS_RATIO = 2

_CW2 = (3, 4, 5, 9, 10, 11, 16, 17, 18)
_CW075 = (15, 7, 12, 8)

_NC, _NS = 2, 16
_NW = _NC * _NS


def _bcast(x, g):
    return x.at[jnp.full((16,), g, jnp.int32)].get(mode="promise_in_bounds")


def _bfly_max(x, lane):
    for k in (1, 2, 4, 8):
        x = jnp.maximum(x, x.at[lane ^ k].get(mode="promise_in_bounds"))
    return x


def _bfly_min(x, lane):
    for k in (1, 2, 4, 8):
        x = jnp.minimum(x, x.at[lane ^ k].get(mode="promise_in_bounds"))
    return x


def _sc_match_body(gtbT_hbm, gtl_hbm, defT_hbm, bti_out, conf_out,
                   def_v, bti_v, conf_v, gtbT_v, gtl_v, *, B, G, PP):
    CH = PP // 16
    BPW = B // _NW
    wid = lax.axis_index("s") * _NC + lax.axis_index("c")
    lane = lax.broadcasted_iota(jnp.int32, (16,), 0)

    pltpu.sync_copy(defT_hbm, def_v)

    for bi in range(BPW):
        b = wid * BPW + bi
        pltpu.sync_copy(gtbT_hbm.at[b], gtbT_v)
        pltpu.sync_copy(gtl_hbm.at[b], gtl_v)

        x1v = gtbT_v[0, pl.ds(0, 16)]
        y1v = gtbT_v[1, pl.ds(0, 16)]
        x2v = gtbT_v[2, pl.ds(0, 16)]
        y2v = gtbT_v[3, pl.ds(0, 16)]
        labv = gtl_v[pl.ds(0, 16)]
        tx1 = [_bcast(x1v, g) for g in range(G)]
        ty1 = [_bcast(y1v, g) for g in range(G)]
        tx2 = [_bcast(x2v, g) for g in range(G)]
        ty2 = [_bcast(y2v, g) for g in range(G)]
        tarea = [(tx2[g] - tx1[g]) * (ty2[g] - ty1[g]) for g in range(G)]

        def chunk_body(i, carry):
            mx = list(carry[:G])
            mi = list(carry[G:])
            s = i * 16
            dx1 = def_v[0, pl.ds(s, 16)]
            dy1 = def_v[1, pl.ds(s, 16)]
            dx2 = def_v[2, pl.ds(s, 16)]
            dy2 = def_v[3, pl.ds(s, 16)]
            darea = (dx2 - dx1) * (dy2 - dy1)
            bov = None
            bti = None
            for g in range(G):
                iw = jnp.maximum(
                    jnp.minimum(dx2, tx2[g]) - jnp.maximum(dx1, tx1[g]), 0.0)
                ih = jnp.maximum(
                    jnp.minimum(dy2, ty2[g]) - jnp.maximum(dy1, ty1[g]), 0.0)
                inter = iw * ih
                ov = inter / (darea + (tarea[g] - inter))
                t2 = ov > mx[g]
                mx[g] = jnp.where(t2, ov, mx[g])
                mi[g] = jnp.where(t2, i, mi[g])
                if g == 0:
                    bov = ov
                    bti = jnp.zeros((16,), jnp.int32)
                else:
                    t = ov > bov
                    bov = jnp.where(t, ov, bov)
                    bti = jnp.where(t, g, bti)
            labm = labv.at[bti].get(mode="promise_in_bounds")
            conf = jnp.where(bov < THRESHOLD, 0, labm)
            bti_v[pl.ds(s, 16)] = bti
            conf_v[pl.ds(s, 16)] = conf
            return tuple(mx) + tuple(mi)

        init = (tuple(jnp.full((16,), -1.0, jnp.float32) for _ in range(G))
                + tuple(jnp.zeros((16,), jnp.int32) for _ in range(G)))
        res = lax.fori_loop(0, CH, chunk_body, init)

        bps = []
        labg = []
        for g in range(G):
            mx = res[g]
            mi = res[G + g]
            m = _bfly_max(mx, lane)
            cand = jnp.where(mx == m, mi * 16 + lane, jnp.int32(0x7FFFFFFF))
            bps.append(_bfly_min(cand, lane))
            labg.append(_bcast(labv, g))

        def force_body(i, c):
            s = i * 16
            linv = s + lane
            bti = bti_v[pl.ds(s, 16)]
            conf = conf_v[pl.ds(s, 16)]
            for g in range(G):
                hit = linv == bps[g]
                bti = jnp.where(hit, g, bti)
                conf = jnp.where(hit, labg[g], conf)
            bti_v[pl.ds(s, 16)] = bti
            conf_v[pl.ds(s, 16)] = conf
            return c

        lax.fori_loop(0, CH, force_body, jnp.int32(0))

        pltpu.sync_copy(bti_v, bti_out.at[b])
        pltpu.sync_copy(conf_v, conf_out.at[b])


def _sc_match(gtbT, gtl, defT, B, G, PP):
    mesh = plsc.VectorSubcoreMesh(core_axis_name="c", subcore_axis_name="s",
                                  num_cores=_NC, num_subcores=_NS)
    return pl.kernel(
        functools.partial(_sc_match_body, B=B, G=G, PP=PP),
        out_type=[jax.ShapeDtypeStruct((B, PP), jnp.int32),
                  jax.ShapeDtypeStruct((B, PP), jnp.int32)],
        mesh=mesh,
        scratch_types=[
            pltpu.VMEM((4, PP), jnp.float32),
            pltpu.VMEM((PP,), jnp.int32),
            pltpu.VMEM((PP,), jnp.int32),
            pltpu.VMEM((4, G), jnp.float32),
            pltpu.VMEM((G,), jnp.int32),
        ],
    )(gtbT, gtl, defT)


def _row_kernel(gtb_ref, def_ref, loc_ref, conf_ref, bti_ref, ct_ref,
                lc_ref, st_ref, *, P, G):
    R, L = def_ref.shape[1], def_ref.shape[2]
    shape = (R, L)
    lin = (lax.broadcasted_iota(jnp.int32, shape, 0) * L
           + lax.broadcasted_iota(jnp.int32, shape, 1))
    valid = lin < P

    dx1 = def_ref[0]
    dy1 = def_ref[1]
    dx2 = def_ref[2]
    dy2 = def_ref[3]
    dw = dx2 - dx1
    dh = dy2 - dy1
    dcx = dx1 + dw * 0.5
    dcy = dy1 + dh * 0.5

    bti = bti_ref[0]
    conf_t = ct_ref[0]

    gx1 = jnp.zeros(shape, jnp.float32)
    gy1 = jnp.zeros(shape, jnp.float32)
    gx2 = jnp.zeros(shape, jnp.float32)
    gy2 = jnp.zeros(shape, jnp.float32)
    for g in range(G):
        hit = bti == g
        gx1 = jnp.where(hit, gtb_ref[0, g, 0], gx1)
        gy1 = jnp.where(hit, gtb_ref[0, g, 1], gy1)
        gx2 = jnp.where(hit, gtb_ref[0, g, 2], gx2)
        gy2 = jnp.where(hit, gtb_ref[0, g, 3], gy2)

    pos = jnp.logical_and(conf_t > 0, valid)
    posf = pos.astype(jnp.float32)

    gw = gx2 - gx1
    gh = gy2 - gy1
    gcx = gx1 + gw * 0.5
    gcy = gy1 + gh * 0.5
    e0 = (gcx - dcx) / (dw * VAR[0] + 1e-8)
    e1 = (gcy - dcy) / (dh * VAR[0] + 1e-8)
    e2 = jnp.log(gw / (dw + 1e-8) + 1e-8) / VAR[1]
    e3 = jnp.log(gh / (dh + 1e-8) + 1e-8) / VAR[1]

    def sl1(x, t):
        d = jnp.abs(x - t)
        return jnp.where(d < 1.0, 0.5 * d * d, d - 0.5)

    sl = (sl1(loc_ref[0, 0], e0) + sl1(loc_ref[0, 1], e1)
          + sl1(loc_ref[0, 2], e2) + sl1(loc_ref[0, 3], e3))

    m = conf_ref[0, 0]
    for c in range(1, NUM_CLASSES):
        m = jnp.maximum(m, conf_ref[0, c])
    s = jnp.exp(conf_ref[0, 0] - m)
    for c in range(1, NUM_CLASSES):
        s = s + jnp.exp(conf_ref[0, c] - m)
    lse = m + jnp.log(s)

    logit_t = conf_ref[0, 0]
    for c in range(1, NUM_CLASSES):
        logit_t = jnp.where(conf_t == c, conf_ref[0, c], logit_t)
    ce = lse - logit_t
    pt = jnp.exp(-ce)
    omp = jnp.maximum(1.0 - pt, 0.0)
    cw = jnp.full(shape, 1.0, jnp.float32)
    for c in _CW2:
        cw = jnp.where(conf_t == c, 2.0, cw)
    for c in _CW075:
        cw = jnp.where(conf_t == c, 0.75, cw)
    cw = jnp.where(conf_t == 0, 0.5, cw)
    focal = ALPHA * omp * jnp.sqrt(omp) * cw * ce

    lc_ref[0] = jnp.where(jnp.logical_and(valid, jnp.logical_not(pos)),
                          focal, 0.0)
    st_ref[0, 0, 0] = jnp.sum(sl * posf)
    st_ref[0, 0, 1] = jnp.sum(posf)
    st_ref[0, 0, 2] = jnp.sum(focal * posf)
    st_ref[0, 0, 3] = 0.0


def _mine_kernel(lc_ref, sl_ref, np_ref, fp_ref, out_ref, *, P):
    B = lc_ref.shape[0]
    npos = np_ref[...]
    k = jnp.minimum(jnp.float32(NEG_POS_RATIO) * npos, jnp.float32(P - 1))
    ki = k.astype(jnp.int32)
    vi = lax.bitcast_convert_type(lc_ref[...], jnp.int32)

    def body(_, lohi):
        lo, hi = lohi
        mid = lo + lax.div(hi - lo, 2)
        c = jnp.sum((vi >= mid).astype(jnp.int32), axis=(1, 2), keepdims=True)
        good = c >= ki
        return jnp.where(good, mid, lo), jnp.where(good, hi, mid)

    lo, _ = lax.fori_loop(
        0, 31, body,
        (jnp.zeros((B, 1, 1), jnp.int32),
         jnp.full((B, 1, 1), 0x7F800000, jnp.int32)))
    vk = lax.bitcast_convert_type(lo, jnp.float32)
    gt_mask = vi > lo
    cnt_gt = jnp.sum(gt_mask.astype(jnp.float32), axis=(1, 2), keepdims=True)
    topk = (jnp.sum(jnp.where(gt_mask, lc_ref[...], 0.0), axis=(1, 2),
                    keepdims=True)
            + (k - cnt_gt) * vk)

    sel_cnt = npos + jnp.minimum(k, jnp.float32(P) - npos)
    loc_loss = jnp.sum(sl_ref[...]) / jnp.sum(npos)
    conf_loss = (jnp.sum(fp_ref[...]) + jnp.sum(topk)) / jnp.sum(sel_cnt)
    out_ref[0] = LAMBDA_LOC * loc_loss + LAMBDA_CONF * conf_loss


def _ssd_loss_hybrid(loc_preds, conf_preds, gt_boxes, gt_labels,
                     default_boxes, interpret=False):
    B, P, C = conf_preds.shape
    G = gt_boxes.shape[1]
    L = 128
    PP = ((P + L - 1) // L) * L
    R = PP // L

    locT = jnp.moveaxis(loc_preds, 2, 1)
    locT = jnp.pad(locT, ((0, 0), (0, 0), (0, PP - P))).reshape(B, 4, R, L)
    confT = jnp.moveaxis(conf_preds, 2, 1)
    confT = jnp.pad(confT, ((0, 0), (0, 0), (0, PP - P))).reshape(B, C, R, L)
    defT = jnp.pad(default_boxes.T, ((0, 0), (0, PP - P)))
    defT4 = defT.reshape(4, R, L)
    gtbT = jnp.moveaxis(gt_boxes, 2, 1)
    gtl = gt_labels.astype(jnp.int32)
    gtb = gt_boxes

    bti, conf_t = _sc_match(gtbT, gtl, defT, B, G, PP)
    bti = bti.reshape(B, R, L)
    conf_t = conf_t.reshape(B, R, L)

    loss_c, stats = pl.pallas_call(
        functools.partial(_row_kernel, P=P, G=G),
        grid=(B,),
        in_specs=[
            pl.BlockSpec((1, G, 4), lambda b: (b, 0, 0),
                         memory_space=pltpu.SMEM),
            pl.BlockSpec((4, R, L), lambda b: (0, 0, 0)),
            pl.BlockSpec((1, 4, R, L), lambda b: (b, 0, 0, 0)),
            pl.BlockSpec((1, C, R, L), lambda b: (b, 0, 0, 0)),
            pl.BlockSpec((1, R, L), lambda b: (b, 0, 0)),
            pl.BlockSpec((1, R, L), lambda b: (b, 0, 0)),
        ],
        out_specs=[
            pl.BlockSpec((1, R, L), lambda b: (b, 0, 0)),
            pl.BlockSpec((1, 1, 4), lambda b: (b, 0, 0),
                         memory_space=pltpu.SMEM),
        ],
        out_shape=[
            jax.ShapeDtypeStruct((B, R, L), jnp.float32),
            jax.ShapeDtypeStruct((B, 1, 4), jnp.float32),
        ],
        interpret=interpret,
    )(gtb, defT4, locT, confT, bti, conf_t)

    sl_sum = stats[:, 0, 0].reshape(B, 1, 1)
    npos = stats[:, 0, 1].reshape(B, 1, 1)
    fpos = stats[:, 0, 2].reshape(B, 1, 1)

    out = pl.pallas_call(
        functools.partial(_mine_kernel, P=P),
        out_specs=pl.BlockSpec(memory_space=pltpu.SMEM),
        out_shape=jax.ShapeDtypeStruct((1,), jnp.float32),
        interpret=interpret,
    )(loss_c, sl_sum, npos, fpos)

    return out[0]


def kernel(loc_preds, conf_preds, gt_boxes, gt_labels, default_boxes):
    return _ssd_loss_hybrid(loc_preds, conf_preds, gt_boxes, gt_labels,
                            default_boxes)
